# Initial kernel scaffold; baseline (speedup 1.0000x reference)
import jax
import jax.numpy as jnp
from jax.experimental import pallas as pl


def kernel(x, rel_embed, edge_index, edge_type, w_in, w_out, w_loop, gamma, beta):
    raise NotImplementedError("write your pallas kernel here")


# speedup vs baseline: 30.2586x; 30.2586x over previous
import functools

import jax
import jax.numpy as jnp
from jax import lax
from jax.experimental import pallas as pl
from jax.experimental.pallas import tpu as pltpu
import jax.experimental.pallas.tpu_sc as plsc

N_ENT = 10000
EMB = 256
NP = 10240
CHUNK = NP // 16
HALF = 80000
LW = 128
NROW = 40
EPT = NROW * LW

_MESH = plsc.VectorSubcoreMesh(
    core_axis_name="c", subcore_axis_name="s", num_cores=2, num_subcores=16)


@functools.partial(
    pl.kernel,
    out_type=jax.ShapeDtypeStruct((2, NP), jnp.float32),
    mesh=_MESH,
    compiler_params=pltpu.CompilerParams(needs_layout_passes=False),
    scratch_types=[
        pltpu.VMEM((NROW, LW), jnp.int32),
        pltpu.VMEM((NROW, LW), jnp.int32),
        pltpu.VMEM((NROW, LW), jnp.float32),
        pltpu.VMEM((NP,), jnp.float32),
        pltpu.VMEM((CHUNK,), jnp.float32),
        pltpu.VMEM((CHUNK,), jnp.float32),
        pltpu.VMEM_SHARED((NP,), jnp.float32),
        pltpu.VMEM_SHARED((NP,), jnp.float32),
    ],
)
def _edge_scalars(row_hbm, col_hbm, out_hbm, row_v, col_v, val_v,
                  dinv_v, cha, chb, sh_deg, sh_t):
    c = lax.axis_index("c")
    s = lax.axis_index("s")

    pltpu.sync_copy(row_hbm.at[c, s], row_v)
    pltpu.sync_copy(col_hbm.at[c, s], col_v)

    def _zero(i, carry):
        cha[pl.ds(i * 16, 16)] = jnp.zeros((16,), jnp.float32)
        return carry
    lax.fori_loop(0, CHUNK // 16, _zero, 0)
    pltpu.sync_copy(cha, sh_deg.at[pl.ds(s * CHUNK, CHUNK)])
    pltpu.sync_copy(cha, sh_t.at[pl.ds(s * CHUNK, CHUNK)])

    def _ones(i, carry):
        val_v[i // 8, pl.ds((i % 8) * 16, 16)] = jnp.ones((16,), jnp.float32)
        return carry
    lax.fori_loop(0, NROW * 8, _ones, 0)
    plsc.subcore_barrier()

    def _p1(j, carry):
        pltpu.sync_copy(val_v.at[j], sh_deg.at[row_v.at[j]], add=True)
        return carry
    lax.fori_loop(0, NROW, _p1, 0)
    plsc.subcore_barrier()

    pltpu.sync_copy(sh_deg.at[pl.ds(s * CHUNK, CHUNK)], cha)

    def _p2(i, carry):
        d = cha[pl.ds(i * 16, 16)]
        y = d * 0.25 + 1.0
        for _ in range(12):
            y = (y + d / y) * 0.5
        cha[pl.ds(i * 16, 16)] = jnp.where(d > 0.5, 1.0 / y, 0.0)
        return carry
    lax.fori_loop(0, CHUNK // 16, _p2, 0)
    pltpu.sync_copy(cha, sh_deg.at[pl.ds(s * CHUNK, CHUNK)])
    plsc.subcore_barrier()

    pltpu.sync_copy(sh_deg, dinv_v)

    def _p3a(i, carry):
        idx = col_v[i // 8, pl.ds((i % 8) * 16, 16)]
        val_v[i // 8, pl.ds((i % 8) * 16, 16)] = plsc.load_gather(dinv_v, [idx])
        return carry
    lax.fori_loop(0, NROW * 8, _p3a, 0)

    def _p3b(j, carry):
        pltpu.sync_copy(val_v.at[j], sh_t.at[row_v.at[j]], add=True)
        return carry
    lax.fori_loop(0, NROW, _p3b, 0)
    plsc.subcore_barrier()

    pltpu.sync_copy(sh_deg.at[pl.ds(s * CHUNK, CHUNK)], cha)
    pltpu.sync_copy(sh_t.at[pl.ds(s * CHUNK, CHUNK)], chb)

    def _p4(i, carry):
        cha[pl.ds(i * 16, 16)] = cha[pl.ds(i * 16, 16)] * chb[pl.ds(i * 16, 16)]
        return carry
    lax.fori_loop(0, CHUNK // 16, _p4, 0)
    pltpu.sync_copy(cha, out_hbm.at[c, pl.ds(s * CHUNK, CHUNK)])


BM = 2000
NB = N_ENT // BM


def _mm_body(x_ref, win_ref, wout_ref, wloop_ref, sin_ref, sout_ref,
             pre_ref, stats_ref, acc):
    i = pl.program_id(0)
    x = x_ref[...]
    hi = jax.lax.Precision.HIGHEST
    pre = (jnp.dot(x, win_ref[...], precision=hi) * sin_ref[...]
           + jnp.dot(x, wout_ref[...], precision=hi) * sout_ref[...]
           + jnp.dot(x, wloop_ref[...], precision=hi)) * (1.0 / 3.0)
    pre_ref[...] = pre

    @pl.when(i == 0)
    def _():
        acc[...] = jnp.zeros_like(acc)

    acc[0:1, :] += jnp.sum(pre, axis=0, keepdims=True)
    acc[1:2, :] += jnp.sum(pre * pre, axis=0, keepdims=True)

    @pl.when(i == NB - 1)
    def _():
        stats_ref[...] = acc[...]


_mm = pl.pallas_call(
    _mm_body,
    grid=(NB,),
    in_specs=[
        pl.BlockSpec((BM, EMB), lambda i: (i, 0)),
        pl.BlockSpec((EMB, EMB), lambda i: (0, 0)),
        pl.BlockSpec((EMB, EMB), lambda i: (0, 0)),
        pl.BlockSpec((EMB, EMB), lambda i: (0, 0)),
        pl.BlockSpec((BM, 1), lambda i: (i, 0)),
        pl.BlockSpec((BM, 1), lambda i: (i, 0)),
    ],
    out_specs=[
        pl.BlockSpec((BM, EMB), lambda i: (i, 0)),
        pl.BlockSpec((2, EMB), lambda i: (0, 0)),
    ],
    out_shape=[
        jax.ShapeDtypeStruct((N_ENT, EMB), jnp.float32),
        jax.ShapeDtypeStruct((2, EMB), jnp.float32),
    ],
    scratch_shapes=[pltpu.VMEM((2, EMB), jnp.float32)],
)


def _bn_body(pre_ref, stats_ref, g_ref, b_ref, o_ref):
    mean = stats_ref[0:1, :] * (1.0 / N_ENT)
    var = stats_ref[1:2, :] * (1.0 / N_ENT) - mean * mean
    o_ref[...] = jnp.tanh((pre_ref[...] - mean) * lax.rsqrt(var + 1e-5)
                          * g_ref[...] + b_ref[...])


_bn = pl.pallas_call(
    _bn_body,
    grid=(NB,),
    in_specs=[
        pl.BlockSpec((BM, EMB), lambda i: (i, 0)),
        pl.BlockSpec((2, EMB), lambda i: (0, 0)),
        pl.BlockSpec((1, EMB), lambda i: (0, 0)),
        pl.BlockSpec((1, EMB), lambda i: (0, 0)),
    ],
    out_specs=pl.BlockSpec((BM, EMB), lambda i: (i, 0)),
    out_shape=jax.ShapeDtypeStruct((N_ENT, EMB), jnp.float32),
)


def kernel(x, rel_embed, edge_index, edge_type, w_in, w_out, w_loop,
           gamma, beta):
    half = edge_index.shape[1] // 2
    row = edge_index[0].astype(jnp.int32)
    col = edge_index[1].astype(jnp.int32)
    npad = 16 * EPT - half
    pad_idx = N_ENT + (jnp.arange(npad, dtype=jnp.int32) % (NP - N_ENT))

    def _prep(a):
        return jnp.concatenate([a, pad_idx]).reshape(16, NROW, LW)

    row_all = jnp.stack([_prep(row[:half]), _prep(row[half:])])
    col_all = jnp.stack([_prep(col[:half]), _prep(col[half:])])
    s_all = _edge_scalars(row_all, col_all)
    s_in = s_all[0, :N_ENT, None]
    s_out = s_all[1, :N_ENT, None]
    pre, stats = _mm(x, w_in, w_out, w_loop, s_in, s_out)
    res = _bn(pre, stats, gamma.reshape(1, EMB), beta.reshape(1, EMB))
    return (res, rel_embed)


# speedup vs baseline: 37.0327x; 1.2239x over previous
import functools

import jax
import jax.numpy as jnp
from jax import lax
from jax.experimental import pallas as pl
from jax.experimental.pallas import tpu as pltpu
import jax.experimental.pallas.tpu_sc as plsc

N_ENT = 10000
EMB = 256
NP = 10240
CHUNK = NP // 16
HALF = 80000
LW = 128
NROW = 40
EPT = NROW * LW

_MESH = plsc.VectorSubcoreMesh(
    core_axis_name="c", subcore_axis_name="s", num_cores=2, num_subcores=16)


@functools.partial(
    pl.kernel,
    out_type=jax.ShapeDtypeStruct((2, NP), jnp.float32),
    mesh=_MESH,
    compiler_params=pltpu.CompilerParams(needs_layout_passes=False),
    scratch_types=[
        pltpu.VMEM((NROW, LW), jnp.int32),
        pltpu.VMEM((NROW, LW), jnp.int32),
        pltpu.VMEM((NROW, LW), jnp.float32),
        pltpu.VMEM((CHUNK,), jnp.float32),
        pltpu.VMEM((CHUNK,), jnp.float32),
        pltpu.VMEM_SHARED((NP,), jnp.float32),
        pltpu.VMEM_SHARED((NP,), jnp.float32),
        pltpu.SemaphoreType.DMA,
    ],
)
def _edge_scalars(idx_hbm, zeros_hbm, ones_hbm, out_hbm,
                  row_v, col_v, val_v, cha, chb, sh_deg, sh_t, sem):
    c = lax.axis_index("c")
    s = lax.axis_index("s")

    pltpu.sync_copy(idx_hbm.at[0, c, s], row_v)
    pltpu.sync_copy(idx_hbm.at[1, c, s], col_v)
    pltpu.sync_copy(ones_hbm, val_v)
    pltpu.sync_copy(zeros_hbm.at[pl.ds(s * CHUNK, CHUNK)],
                    sh_deg.at[pl.ds(s * CHUNK, CHUNK)])
    pltpu.sync_copy(zeros_hbm.at[pl.ds(s * CHUNK, CHUNK)],
                    sh_t.at[pl.ds(s * CHUNK, CHUNK)])
    plsc.subcore_barrier()

    def _p1(k, carry):
        descs = [pltpu.async_copy(val_v.at[k * 8 + i],
                                  sh_deg.at[row_v.at[k * 8 + i]], sem,
                                  add=True) for i in range(8)]
        for dsc in descs:
            dsc.wait()
        return carry
    lax.fori_loop(0, NROW // 8, _p1, 0)
    plsc.subcore_barrier()

    pltpu.sync_copy(sh_deg.at[pl.ds(s * CHUNK, CHUNK)], cha)

    def _p2(i, carry):
        d = cha[pl.ds(i * 16, 16)]
        c1 = d >= 65536.0
        d1 = jnp.where(c1, d * (1.0 / 65536.0), d)
        r1 = jnp.where(c1, 1.0 / 256.0, 1.0)
        c2 = d1 >= 256.0
        d2 = jnp.where(c2, d1 * (1.0 / 256.0), d1)
        r2 = jnp.where(c2, 1.0 / 16.0, 1.0)
        c3 = d2 >= 16.0
        d3 = jnp.where(c3, d2 * (1.0 / 16.0), d2)
        r3 = jnp.where(c3, 0.25, 1.0)
        y = d3 * 0.25 + 0.97
        y = (y + d3 / y) * 0.5
        y = (y + d3 / y) * 0.5
        y = (y + d3 / y) * 0.5
        y = (y + d3 / y) * 0.5
        dinv = (r1 * r2 * r3) / y
        cha[pl.ds(i * 16, 16)] = jnp.where(d > 0.5, dinv, 0.0)
        return carry
    lax.fori_loop(0, CHUNK // 16, _p2, 0)
    pltpu.sync_copy(cha, sh_deg.at[pl.ds(s * CHUNK, CHUNK)])
    plsc.subcore_barrier()

    def _p3a(k, carry):
        descs = [pltpu.async_copy(sh_deg.at[col_v.at[k * 8 + i]],
                                  val_v.at[k * 8 + i], sem)
                 for i in range(8)]
        for dsc in descs:
            dsc.wait()
        return carry
    lax.fori_loop(0, NROW // 8, _p3a, 0)

    def _p3b(k, carry):
        descs = [pltpu.async_copy(val_v.at[k * 8 + i],
                                  sh_t.at[row_v.at[k * 8 + i]], sem,
                                  add=True) for i in range(8)]
        for dsc in descs:
            dsc.wait()
        return carry
    lax.fori_loop(0, NROW // 8, _p3b, 0)
    plsc.subcore_barrier()

    pltpu.sync_copy(sh_deg.at[pl.ds(s * CHUNK, CHUNK)], cha)
    pltpu.sync_copy(sh_t.at[pl.ds(s * CHUNK, CHUNK)], chb)

    def _p4(i, carry):
        cha[pl.ds(i * 16, 16)] = cha[pl.ds(i * 16, 16)] * chb[pl.ds(i * 16, 16)]
        return carry
    lax.fori_loop(0, CHUNK // 16, _p4, 0)
    pltpu.sync_copy(cha, out_hbm.at[c, pl.ds(s * CHUNK, CHUNK)])


BM = 2000
NB = N_ENT // BM


def _fused_body(x_ref, win_ref, wout_ref, wloop_ref, sin_ref, sout_ref,
                g_ref, b_ref, o_ref, pre_scr, acc_scr, stat_scr):
    p = pl.program_id(0)
    j = pl.program_id(1)

    @pl.when(p == 0)
    def _():
        x = x_ref[...]
        hi = jax.lax.Precision.HIGHEST
        pre = (jnp.dot(x, win_ref[...], precision=hi) * sin_ref[...]
               + jnp.dot(x, wout_ref[...], precision=hi) * sout_ref[...]
               + jnp.dot(x, wloop_ref[...], precision=hi)) * (1.0 / 3.0)
        pre_scr[pl.ds(j * BM, BM), :] = pre

        @pl.when(j == 0)
        def _():
            acc_scr[...] = jnp.zeros_like(acc_scr)

        acc_scr[0:1, :] += jnp.sum(pre, axis=0, keepdims=True)
        acc_scr[1:2, :] += jnp.sum(pre * pre, axis=0, keepdims=True)

        @pl.when(j == NB - 1)
        def _():
            mean = acc_scr[0:1, :] * (1.0 / N_ENT)
            var = acc_scr[1:2, :] * (1.0 / N_ENT) - mean * mean
            a = lax.rsqrt(var + 1e-5) * g_ref[...]
            stat_scr[0:1, :] = a
            stat_scr[1:2, :] = b_ref[...] - mean * a

    @pl.when(p == 1)
    def _():
        pre = pre_scr[pl.ds(j * BM, BM), :]
        o_ref[...] = jnp.tanh(pre * stat_scr[0:1, :] + stat_scr[1:2, :])


_fused = pl.pallas_call(
    _fused_body,
    grid=(2, NB),
    in_specs=[
        pl.BlockSpec((BM, EMB), lambda p, j: ((1 - p) * j, 0)),
        pl.BlockSpec((EMB, EMB), lambda p, j: (0, 0)),
        pl.BlockSpec((EMB, EMB), lambda p, j: (0, 0)),
        pl.BlockSpec((EMB, EMB), lambda p, j: (0, 0)),
        pl.BlockSpec((BM, 1), lambda p, j: ((1 - p) * j, 0)),
        pl.BlockSpec((BM, 1), lambda p, j: ((1 - p) * j, 0)),
        pl.BlockSpec((1, EMB), lambda p, j: (0, 0)),
        pl.BlockSpec((1, EMB), lambda p, j: (0, 0)),
    ],
    out_specs=pl.BlockSpec((BM, EMB), lambda p, j: (p * j, 0)),
    out_shape=jax.ShapeDtypeStruct((N_ENT, EMB), jnp.float32),
    scratch_shapes=[
        pltpu.VMEM((N_ENT, EMB), jnp.float32),
        pltpu.VMEM((2, EMB), jnp.float32),
        pltpu.VMEM((2, EMB), jnp.float32),
    ],
)


def kernel(x, rel_embed, edge_index, edge_type, w_in, w_out, w_loop,
           gamma, beta):
    half = edge_index.shape[1] // 2
    ei = edge_index.astype(jnp.int32)
    npad = 16 * EPT - half
    pad_idx = N_ENT + (jnp.arange(npad, dtype=jnp.int32) % (NP - N_ENT))
    halves = jnp.stack([ei[:, :half], ei[:, half:]], axis=1)
    pad_blk = jnp.broadcast_to(pad_idx, (2, 2, npad))
    idx_all = jnp.concatenate([halves, pad_blk], axis=-1).reshape(
        2, 2, 16, NROW, LW)
    zeros = jnp.zeros((NP,), jnp.float32)
    ones = jnp.ones((NROW, LW), jnp.float32)
    s_all = _edge_scalars(idx_all, zeros, ones)
    s_in = s_all[0, :N_ENT, None]
    s_out = s_all[1, :N_ENT, None]
    res = _fused(x, w_in, w_out, w_loop, s_in, s_out,
                 gamma.reshape(1, EMB), beta.reshape(1, EMB))
    return (res, rel_embed)


# speedup vs baseline: 51.3557x; 1.3868x over previous
import functools

import jax
import jax.numpy as jnp
from jax import lax
from jax.experimental import pallas as pl
from jax.experimental.pallas import tpu as pltpu
import jax.experimental.pallas.tpu_sc as plsc

N_ENT = 10000
EMB = 256
NP = 10240
CHUNK = NP // 16
HALF = 80000
LW = 128
NROW = 40
EPT = NROW * LW

_MESH = plsc.VectorSubcoreMesh(
    core_axis_name="c", subcore_axis_name="s", num_cores=2, num_subcores=16)


@functools.partial(
    pl.kernel,
    out_type=jax.ShapeDtypeStruct((2, NP), jnp.float32),
    mesh=_MESH,
    compiler_params=pltpu.CompilerParams(needs_layout_passes=False),
    scratch_types=[
        pltpu.VMEM((NROW, LW), jnp.int32),
        pltpu.VMEM((NROW, LW), jnp.int32),
        pltpu.VMEM((NROW, LW), jnp.float32),
        pltpu.VMEM((CHUNK,), jnp.float32),
        pltpu.VMEM((CHUNK,), jnp.float32),
        pltpu.VMEM_SHARED((NP,), jnp.float32),
        pltpu.VMEM_SHARED((NP,), jnp.float32),
        pltpu.SemaphoreType.DMA,
    ],
)
def _edge_scalars(idx_hbm, zeros_hbm, ones_hbm, out_hbm,
                  row_v, col_v, val_v, cha, chb, sh_deg, sh_t, sem):
    c = lax.axis_index("c")
    s = lax.axis_index("s")

    descs = [
        pltpu.async_copy(idx_hbm.at[0, c, s], row_v, sem),
        pltpu.async_copy(idx_hbm.at[1, c, s], col_v, sem),
        pltpu.async_copy(ones_hbm, val_v, sem),
        pltpu.async_copy(zeros_hbm.at[pl.ds(s * CHUNK, CHUNK)],
                         sh_deg.at[pl.ds(s * CHUNK, CHUNK)], sem),
        pltpu.async_copy(zeros_hbm.at[pl.ds(s * CHUNK, CHUNK)],
                         sh_t.at[pl.ds(s * CHUNK, CHUNK)], sem),
    ]
    for dsc in descs:
        dsc.wait()
    plsc.subcore_barrier()

    def _p1(k, carry):
        descs = [pltpu.async_copy(val_v.at[k * 20 + i],
                                  sh_deg.at[row_v.at[k * 20 + i]], sem,
                                  add=True) for i in range(20)]
        for dsc in descs:
            dsc.wait()
        return carry
    lax.fori_loop(0, NROW // 20, _p1, 0)
    plsc.subcore_barrier()

    pltpu.sync_copy(sh_deg.at[pl.ds(s * CHUNK, CHUNK)], cha)

    def _p2(i, carry):
        d = cha[pl.ds(i * 16, 16)]
        c1 = d >= 65536.0
        d1 = jnp.where(c1, d * (1.0 / 65536.0), d)
        r1 = jnp.where(c1, 1.0 / 256.0, 1.0)
        c2 = d1 >= 256.0
        d2 = jnp.where(c2, d1 * (1.0 / 256.0), d1)
        r2 = jnp.where(c2, 1.0 / 16.0, 1.0)
        c3 = d2 >= 16.0
        d3 = jnp.where(c3, d2 * (1.0 / 16.0), d2)
        r3 = jnp.where(c3, 0.25, 1.0)
        y = d3 * 0.25 + 0.97
        y = (y + d3 / y) * 0.5
        y = (y + d3 / y) * 0.5
        y = (y + d3 / y) * 0.5
        y = (y + d3 / y) * 0.5
        dinv = (r1 * r2 * r3) / y
        cha[pl.ds(i * 16, 16)] = jnp.where(d > 0.5, dinv, 0.0)
        return carry
    lax.fori_loop(0, CHUNK // 16, _p2, 0)
    pltpu.sync_copy(cha, sh_deg.at[pl.ds(s * CHUNK, CHUNK)])
    plsc.subcore_barrier()

    def _p3a(k, carry):
        descs = [pltpu.async_copy(sh_deg.at[col_v.at[k * 20 + i]],
                                  val_v.at[k * 20 + i], sem)
                 for i in range(20)]
        for dsc in descs:
            dsc.wait()
        return carry
    lax.fori_loop(0, NROW // 20, _p3a, 0)

    def _p3b(k, carry):
        descs = [pltpu.async_copy(val_v.at[k * 20 + i],
                                  sh_t.at[row_v.at[k * 20 + i]], sem,
                                  add=True) for i in range(20)]
        for dsc in descs:
            dsc.wait()
        return carry
    lax.fori_loop(0, NROW // 20, _p3b, 0)
    plsc.subcore_barrier()

    pltpu.sync_copy(sh_deg.at[pl.ds(s * CHUNK, CHUNK)], cha)
    pltpu.sync_copy(sh_t.at[pl.ds(s * CHUNK, CHUNK)], chb)

    def _p4(i, carry):
        cha[pl.ds(i * 16, 16)] = cha[pl.ds(i * 16, 16)] * chb[pl.ds(i * 16, 16)]
        return carry
    lax.fori_loop(0, CHUNK // 16, _p4, 0)
    pltpu.sync_copy(cha, out_hbm.at[c, pl.ds(s * CHUNK, CHUNK)])


BM = 2000
NB = N_ENT // BM


def _fused_body(x_ref, win_ref, wout_ref, wloop_ref, sin_ref, sout_ref,
                g_ref, b_ref, o_ref, pre_scr, acc_scr, stat_scr):
    p = pl.program_id(0)
    j = pl.program_id(1)

    @pl.when(p == 0)
    def _():
        x = x_ref[...]
        pre = (jnp.dot(x, win_ref[...]) * sin_ref[...]
               + jnp.dot(x, wout_ref[...]) * sout_ref[...]
               + jnp.dot(x, wloop_ref[...])) * (1.0 / 3.0)
        pre_scr[pl.ds(j * BM, BM), :] = pre

        @pl.when(j == 0)
        def _():
            acc_scr[...] = jnp.zeros_like(acc_scr)

        acc_scr[0:1, :] += jnp.sum(pre, axis=0, keepdims=True)
        acc_scr[1:2, :] += jnp.sum(pre * pre, axis=0, keepdims=True)

        @pl.when(j == NB - 1)
        def _():
            mean = acc_scr[0:1, :] * (1.0 / N_ENT)
            var = acc_scr[1:2, :] * (1.0 / N_ENT) - mean * mean
            a = lax.rsqrt(var + 1e-5) * g_ref[...]
            stat_scr[0:1, :] = a
            stat_scr[1:2, :] = b_ref[...] - mean * a

    @pl.when(p == 1)
    def _():
        pre = pre_scr[pl.ds(j * BM, BM), :]
        o_ref[...] = jnp.tanh(pre * stat_scr[0:1, :] + stat_scr[1:2, :])


_fused = pl.pallas_call(
    _fused_body,
    grid=(2, NB),
    in_specs=[
        pl.BlockSpec((BM, EMB), lambda p, j: ((1 - p) * j, 0)),
        pl.BlockSpec((EMB, EMB), lambda p, j: (0, 0)),
        pl.BlockSpec((EMB, EMB), lambda p, j: (0, 0)),
        pl.BlockSpec((EMB, EMB), lambda p, j: (0, 0)),
        pl.BlockSpec((BM, 1), lambda p, j: ((1 - p) * j, 0)),
        pl.BlockSpec((BM, 1), lambda p, j: ((1 - p) * j, 0)),
        pl.BlockSpec((1, EMB), lambda p, j: (0, 0)),
        pl.BlockSpec((1, EMB), lambda p, j: (0, 0)),
    ],
    out_specs=pl.BlockSpec((BM, EMB), lambda p, j: (p * j, 0)),
    out_shape=jax.ShapeDtypeStruct((N_ENT, EMB), jnp.float32),
    scratch_shapes=[
        pltpu.VMEM((N_ENT, EMB), jnp.float32),
        pltpu.VMEM((2, EMB), jnp.float32),
        pltpu.VMEM((2, EMB), jnp.float32),
    ],
)


def kernel(x, rel_embed, edge_index, edge_type, w_in, w_out, w_loop,
           gamma, beta):
    half = edge_index.shape[1] // 2
    ei = edge_index.astype(jnp.int32)
    npad = 16 * EPT - half
    pad_idx = N_ENT + (jnp.arange(npad, dtype=jnp.int32) % (NP - N_ENT))
    halves = jnp.stack([ei[:, :half], ei[:, half:]], axis=1)
    pad_blk = jnp.broadcast_to(pad_idx, (2, 2, npad))
    idx_all = jnp.concatenate([halves, pad_blk], axis=-1).reshape(
        2, 2, 16, NROW, LW)
    zeros = jnp.zeros((NP,), jnp.float32)
    ones = jnp.ones((NROW, LW), jnp.float32)
    s_all = _edge_scalars(idx_all, zeros, ones)
    s_in = s_all[0, :N_ENT, None]
    s_out = s_all[1, :N_ENT, None]
    res = _fused(x, w_in, w_out, w_loop, s_in, s_out,
                 gamma.reshape(1, EMB), beta.reshape(1, EMB))
    return (res, rel_embed)


# speedup vs baseline: 53.5429x; 1.0426x over previous
import functools

import numpy as np
import jax
import jax.numpy as jnp
from jax import lax
from jax.experimental import pallas as pl
from jax.experimental.pallas import tpu as pltpu
import jax.experimental.pallas.tpu_sc as plsc

N_ENT = 10000
EMB = 256
NP = 10240
CHUNK = NP // 16
HALF = 80000
EPT = 5120
NPAD = 16 * EPT - HALF

_PAD_BLK = np.broadcast_to(
    (N_ENT + (np.arange(NPAD, dtype=np.int32) % (NP - N_ENT))), (2, 2, NPAD))
_ZEROS = np.zeros((NP,), np.float32)
_ONES = np.ones((EPT,), np.float32)

_MESH = plsc.VectorSubcoreMesh(
    core_axis_name="c", subcore_axis_name="s", num_cores=2, num_subcores=16)


@functools.partial(
    pl.kernel,
    out_type=jax.ShapeDtypeStruct((2, NP), jnp.float32),
    mesh=_MESH,
    compiler_params=pltpu.CompilerParams(needs_layout_passes=False),
    scratch_types=[
        pltpu.VMEM((EPT,), jnp.int32),
        pltpu.VMEM((EPT,), jnp.int32),
        pltpu.VMEM((EPT,), jnp.float32),
        pltpu.VMEM((CHUNK,), jnp.float32),
        pltpu.VMEM((CHUNK,), jnp.float32),
        pltpu.VMEM_SHARED((NP,), jnp.float32),
        pltpu.VMEM_SHARED((NP,), jnp.float32),
        pltpu.SemaphoreType.DMA,
    ],
)
def _edge_scalars(idx_hbm, zeros_hbm, ones_hbm, out_hbm,
                  row_v, col_v, val_v, cha, chb, sh_deg, sh_t, sem):
    c = lax.axis_index("c")
    s = lax.axis_index("s")

    descs = [
        pltpu.async_copy(idx_hbm.at[0, c, s], row_v, sem),
        pltpu.async_copy(idx_hbm.at[1, c, s], col_v, sem),
        pltpu.async_copy(ones_hbm, val_v, sem),
        pltpu.async_copy(zeros_hbm.at[pl.ds(s * CHUNK, CHUNK)],
                         sh_deg.at[pl.ds(s * CHUNK, CHUNK)], sem),
        pltpu.async_copy(zeros_hbm.at[pl.ds(s * CHUNK, CHUNK)],
                         sh_t.at[pl.ds(s * CHUNK, CHUNK)], sem),
    ]
    for dsc in descs:
        dsc.wait()
    plsc.subcore_barrier()

    pltpu.sync_copy(val_v, sh_deg.at[row_v], add=True)
    plsc.subcore_barrier()

    pltpu.sync_copy(sh_deg.at[pl.ds(s * CHUNK, CHUNK)], cha)

    def _p2(i, carry):
        d = cha[pl.ds(i * 16, 16)]
        c1 = d >= 65536.0
        d1 = jnp.where(c1, d * (1.0 / 65536.0), d)
        r1 = jnp.where(c1, 1.0 / 256.0, 1.0)
        c2 = d1 >= 256.0
        d2 = jnp.where(c2, d1 * (1.0 / 256.0), d1)
        r2 = jnp.where(c2, 1.0 / 16.0, 1.0)
        c3 = d2 >= 16.0
        d3 = jnp.where(c3, d2 * (1.0 / 16.0), d2)
        r3 = jnp.where(c3, 0.25, 1.0)
        y = d3 * 0.25 + 0.97
        y = (y + d3 / y) * 0.5
        y = (y + d3 / y) * 0.5
        y = (y + d3 / y) * 0.5
        y = (y + d3 / y) * 0.5
        dinv = (r1 * r2 * r3) / y
        cha[pl.ds(i * 16, 16)] = jnp.where(d > 0.5, dinv, 0.0)
        return carry
    lax.fori_loop(0, CHUNK // 16, _p2, 0)
    pltpu.sync_copy(cha, sh_deg.at[pl.ds(s * CHUNK, CHUNK)])
    plsc.subcore_barrier()

    pltpu.sync_copy(sh_deg.at[col_v], val_v)
    pltpu.sync_copy(val_v, sh_t.at[row_v], add=True)
    plsc.subcore_barrier()

    da = pltpu.async_copy(sh_deg.at[pl.ds(s * CHUNK, CHUNK)], cha, sem)
    db = pltpu.async_copy(sh_t.at[pl.ds(s * CHUNK, CHUNK)], chb, sem)
    da.wait()
    db.wait()

    def _p4(i, carry):
        cha[pl.ds(i * 16, 16)] = cha[pl.ds(i * 16, 16)] * chb[pl.ds(i * 16, 16)]
        return carry
    lax.fori_loop(0, CHUNK // 16, _p4, 0)
    pltpu.sync_copy(cha, out_hbm.at[c, pl.ds(s * CHUNK, CHUNK)])


BM = 2000
NB = N_ENT // BM


def _fused_body(x_ref, win_ref, wout_ref, wloop_ref, sin_ref, sout_ref,
                g_ref, b_ref, o_ref, pre_scr, acc_scr, stat_scr):
    p = pl.program_id(0)
    j = pl.program_id(1)

    @pl.when(p == 0)
    def _():
        x = x_ref[...]
        pre = (jnp.dot(x, win_ref[...]) * sin_ref[0, :, :]
               + jnp.dot(x, wout_ref[...]) * sout_ref[0, :, :]
               + jnp.dot(x, wloop_ref[...])) * (1.0 / 3.0)
        pre_scr[pl.ds(j * BM, BM), :] = pre

        @pl.when(j == 0)
        def _():
            acc_scr[...] = jnp.zeros_like(acc_scr)

        acc_scr[0:1, :] += jnp.sum(pre, axis=0, keepdims=True)
        acc_scr[1:2, :] += jnp.sum(pre * pre, axis=0, keepdims=True)

        @pl.when(j == NB - 1)
        def _():
            mean = acc_scr[0:1, :] * (1.0 / N_ENT)
            var = acc_scr[1:2, :] * (1.0 / N_ENT) - mean * mean
            a = lax.rsqrt(var + 1e-5) * g_ref[...]
            stat_scr[0:1, :] = a
            stat_scr[1:2, :] = b_ref[...] - mean * a

    @pl.when(p == 1)
    def _():
        pre = pre_scr[pl.ds(j * BM, BM), :]
        o_ref[...] = jnp.tanh(pre * stat_scr[0:1, :] + stat_scr[1:2, :])


_fused = pl.pallas_call(
    _fused_body,
    grid=(2, NB),
    in_specs=[
        pl.BlockSpec((BM, EMB), lambda p, j: ((1 - p) * j, 0)),
        pl.BlockSpec((EMB, EMB), lambda p, j: (0, 0)),
        pl.BlockSpec((EMB, EMB), lambda p, j: (0, 0)),
        pl.BlockSpec((EMB, EMB), lambda p, j: (0, 0)),
        pl.BlockSpec((1, BM, 1), lambda p, j: (0, (1 - p) * j, 0)),
        pl.BlockSpec((1, BM, 1), lambda p, j: (1, (1 - p) * j, 0)),
        pl.BlockSpec((1, EMB), lambda p, j: (0, 0)),
        pl.BlockSpec((1, EMB), lambda p, j: (0, 0)),
    ],
    out_specs=pl.BlockSpec((BM, EMB), lambda p, j: (p * j, 0)),
    out_shape=jax.ShapeDtypeStruct((N_ENT, EMB), jnp.float32),
    scratch_shapes=[
        pltpu.VMEM((N_ENT, EMB), jnp.float32),
        pltpu.VMEM((2, EMB), jnp.float32),
        pltpu.VMEM((2, EMB), jnp.float32),
    ],
)


def kernel(x, rel_embed, edge_index, edge_type, w_in, w_out, w_loop,
           gamma, beta):
    half = edge_index.shape[1] // 2
    ei = edge_index.astype(jnp.int32)
    halves = jnp.stack([ei[:, :half], ei[:, half:]], axis=1)
    idx_all = jnp.concatenate([halves, jnp.asarray(_PAD_BLK)], axis=-1)
    idx_all = idx_all.reshape(2, 2, 16, EPT)
    s_all = _edge_scalars(idx_all, jnp.asarray(_ZEROS), jnp.asarray(_ONES))
    s_all = s_all.reshape(2, NP, 1)
    res = _fused(x, w_in, w_out, w_loop, s_all, s_all,
                 gamma.reshape(1, EMB), beta.reshape(1, EMB))
    return (res, rel_embed)


# speedup vs baseline: 59.6621x; 1.1143x over previous
import functools

import numpy as np
import jax
import jax.numpy as jnp
from jax import lax
from jax.experimental import pallas as pl
from jax.experimental.pallas import tpu as pltpu
import jax.experimental.pallas.tpu_sc as plsc

N_ENT = 10000
EMB = 256
NP = 10240
CHUNK = NP // 16
HALF = 80000
EPT = 5120
NREAL15 = EPT - (16 * EPT - HALF)

_ZEROS = np.zeros((NP,), np.float32)
_ONES = np.ones((EPT,), np.float32)

_MESH = plsc.VectorSubcoreMesh(
    core_axis_name="c", subcore_axis_name="s", num_cores=2, num_subcores=16)


@functools.partial(
    pl.kernel,
    out_type=jax.ShapeDtypeStruct((8, NP), jnp.float32),
    mesh=_MESH,
    compiler_params=pltpu.CompilerParams(needs_layout_passes=False),
    scratch_types=[
        pltpu.VMEM((EPT,), jnp.int32),
        pltpu.VMEM((EPT,), jnp.int32),
        pltpu.VMEM((EPT,), jnp.float32),
        pltpu.VMEM((CHUNK,), jnp.float32),
        pltpu.VMEM((CHUNK,), jnp.float32),
        pltpu.VMEM_SHARED((NP,), jnp.float32),
        pltpu.VMEM_SHARED((NP,), jnp.float32),
        pltpu.SemaphoreType.DMA,
    ],
)
def _edge_scalars(ei_hbm, zeros_hbm, ones_hbm, out_hbm,
                  row_v, col_v, val_v, cha, chb, sh_deg, sh_t, sem):
    c = lax.axis_index("c")
    s = lax.axis_index("s")

    descs = [
        pltpu.async_copy(ei_hbm.at[0, c, s], row_v, sem),
        pltpu.async_copy(ei_hbm.at[1, c, s], col_v, sem),
        pltpu.async_copy(ones_hbm, val_v, sem),
        pltpu.async_copy(zeros_hbm.at[pl.ds(s * CHUNK, CHUNK)],
                         sh_deg.at[pl.ds(s * CHUNK, CHUNK)], sem),
        pltpu.async_copy(zeros_hbm.at[pl.ds(s * CHUNK, CHUNK)],
                         sh_t.at[pl.ds(s * CHUNK, CHUNK)], sem),
    ]
    for dsc in descs:
        dsc.wait()

    @pl.when(s == 15)
    def _():
        lanes = lax.iota(jnp.int32, 16)

        def _respread(k, carry):
            pad = N_ENT + ((k * 16 + lanes) % (NP - N_ENT))
            row_v[pl.ds(NREAL15 + k * 16, 16)] = pad
            col_v[pl.ds(NREAL15 + k * 16, 16)] = pad
            return carry
        lax.fori_loop(0, (EPT - NREAL15) // 16, _respread, 0)
    plsc.subcore_barrier()

    pltpu.sync_copy(val_v, sh_deg.at[row_v], add=True)
    plsc.subcore_barrier()

    pltpu.sync_copy(sh_deg.at[pl.ds(s * CHUNK, CHUNK)], cha)

    def _p2(i, carry):
        d = cha[pl.ds(i * 16, 16)]
        c1 = d >= 65536.0
        d1 = jnp.where(c1, d * (1.0 / 65536.0), d)
        r1 = jnp.where(c1, 1.0 / 256.0, 1.0)
        c2 = d1 >= 256.0
        d2 = jnp.where(c2, d1 * (1.0 / 256.0), d1)
        r2 = jnp.where(c2, 1.0 / 16.0, 1.0)
        c3 = d2 >= 16.0
        d3 = jnp.where(c3, d2 * (1.0 / 16.0), d2)
        r3 = jnp.where(c3, 0.25, 1.0)
        y = d3 * 0.25 + 0.97
        y = (y + d3 / y) * 0.5
        y = (y + d3 / y) * 0.5
        y = (y + d3 / y) * 0.5
        y = (y + d3 / y) * 0.5
        dinv = (r1 * r2 * r3) / y
        cha[pl.ds(i * 16, 16)] = jnp.where(d > 0.5, dinv, 0.0)
        return carry
    lax.fori_loop(0, CHUNK // 16, _p2, 0)
    pltpu.sync_copy(cha, sh_deg.at[pl.ds(s * CHUNK, CHUNK)])
    plsc.subcore_barrier()

    pltpu.sync_copy(sh_deg.at[col_v], val_v)
    pltpu.sync_copy(val_v, sh_t.at[row_v], add=True)
    plsc.subcore_barrier()

    da = pltpu.async_copy(sh_deg.at[pl.ds(s * CHUNK, CHUNK)], cha, sem)
    db = pltpu.async_copy(sh_t.at[pl.ds(s * CHUNK, CHUNK)], chb, sem)
    da.wait()
    db.wait()

    def _p4(i, carry):
        cha[pl.ds(i * 16, 16)] = cha[pl.ds(i * 16, 16)] * chb[pl.ds(i * 16, 16)]
        return carry
    lax.fori_loop(0, CHUNK // 16, _p4, 0)
    pltpu.sync_copy(cha, out_hbm.at[c, pl.ds(s * CHUNK, CHUNK)])


BM = 2000
NB = N_ENT // BM


def _fused_body(x_ref, win_ref, wout_ref, wloop_ref, s_ref,
                g_ref, b_ref, o_ref, pre_scr, scol_scr, acc_scr, stat_scr):
    p = pl.program_id(0)
    j = pl.program_id(1)

    @pl.when(p == 0)
    def _():
        @pl.when(j == 0)
        def _():
            scol_scr[...] = jnp.swapaxes(s_ref[...], 0, 1)
            acc_scr[...] = jnp.zeros_like(acc_scr)

        x = x_ref[...]
        scol = scol_scr[pl.ds(j * BM, BM), :]
        pre = (jnp.dot(x, win_ref[...]) * scol[:, 0:1]
               + jnp.dot(x, wout_ref[...]) * scol[:, 1:2]
               + jnp.dot(x, wloop_ref[...])) * (1.0 / 3.0)
        pre_scr[pl.ds(j * BM, BM), :] = pre

        acc_scr[0:1, :] += jnp.sum(pre, axis=0, keepdims=True)
        acc_scr[1:2, :] += jnp.sum(pre * pre, axis=0, keepdims=True)

        @pl.when(j == NB - 1)
        def _():
            mean = acc_scr[0:1, :] * (1.0 / N_ENT)
            var = acc_scr[1:2, :] * (1.0 / N_ENT) - mean * mean
            a = lax.rsqrt(var + 1e-5) * g_ref[...]
            stat_scr[0:1, :] = a
            stat_scr[1:2, :] = b_ref[...] - mean * a

    @pl.when(p == 1)
    def _():
        pre = pre_scr[pl.ds(j * BM, BM), :]
        o_ref[...] = jnp.tanh(pre * stat_scr[0:1, :] + stat_scr[1:2, :])


_fused = pl.pallas_call(
    _fused_body,
    grid=(2, NB),
    in_specs=[
        pl.BlockSpec((BM, EMB), lambda p, j: ((1 - p) * j, 0)),
        pl.BlockSpec((EMB, EMB), lambda p, j: (0, 0)),
        pl.BlockSpec((EMB, EMB), lambda p, j: (0, 0)),
        pl.BlockSpec((EMB, EMB), lambda p, j: (0, 0)),
        pl.BlockSpec((8, NP), lambda p, j: (0, 0)),
        pl.BlockSpec((1, EMB), lambda p, j: (0, 0)),
        pl.BlockSpec((1, EMB), lambda p, j: (0, 0)),
    ],
    out_specs=pl.BlockSpec((BM, EMB), lambda p, j: (p * j, 0)),
    out_shape=jax.ShapeDtypeStruct((N_ENT, EMB), jnp.float32),
    scratch_shapes=[
        pltpu.VMEM((N_ENT, EMB), jnp.float32),
        pltpu.VMEM((NP, 8), jnp.float32),
        pltpu.VMEM((2, EMB), jnp.float32),
        pltpu.VMEM((2, EMB), jnp.float32),
    ],
)


def kernel(x, rel_embed, edge_index, edge_type, w_in, w_out, w_loop,
           gamma, beta):
    ei = edge_index.astype(jnp.int32).reshape(2, 2, HALF)
    ei = jnp.pad(ei, ((0, 0), (0, 0), (0, 16 * EPT - HALF)),
                 constant_values=N_ENT).reshape(2, 2, 16, EPT)
    s_all = _edge_scalars(ei, jnp.asarray(_ZEROS), jnp.asarray(_ONES))
    res = _fused(x, w_in, w_out, w_loop, s_all,
                 gamma.reshape(1, EMB), beta.reshape(1, EMB))
    return (res, rel_embed)
